# baseline (speedup 1.0000x reference)
import functools

import jax
import jax.numpy as jnp
import numpy as np
from jax import lax
from jax.experimental import pallas as pl
from jax.experimental.pallas import tpu as pltpu
from jax.experimental.pallas import tpu_sc as plsc

BATCH = 16384
VOCAB = 1024
SNR = 10.0
_A = float(np.sqrt(10 ** (SNR / 10)))
_SCALE = (2.0 * _A) / (VOCAB - 1.0)

ROWS = 512
SC_ROWS = 3584
TC_ROWS = BATCH - SC_ROWS

_KS0 = np.uint32(0)
_KS1 = np.uint32(42)
_KS2 = np.uint32(int(_KS0) ^ int(_KS1) ^ 0x1BD11BDA)
_R0 = (13, 15, 26, 6)
_R1 = (17, 29, 16, 24)
_TINY = np.float32(np.finfo(np.float32).tiny)


def _rotl(x, r):
    return (x << np.uint32(r)) | (
        lax.shift_right_logical(x, jnp.full_like(x, np.uint32(32 - r)))
    )


def _round4(x0, x1, rots):
    for r in rots:
        x0 = x0 + x1
        x1 = _rotl(x1, r) ^ x0
    return x0, x1


def _threefry_fold(lin):
    x1i = lin + _KS1
    x0 = x1i
    x1 = _rotl(x1i, 13) ^ x1i
    x0, x1 = _round4(x0, x1, _R0[1:])
    x0, x1 = x0 + _KS1, x1 + (_KS2 + np.uint32(1))
    x0, x1 = _round4(x0, x1, _R1)
    x0, x1 = x0 + _KS2, x1 + (_KS0 + np.uint32(2))
    x0, x1 = _round4(x0, x1, _R0)
    x0, x1 = x0 + _KS0, x1 + (_KS1 + np.uint32(3))
    x0, x1 = _round4(x0, x1, _R1)
    x0, x1 = x0 + _KS1, x1 + (_KS2 + np.uint32(4))
    x0, x1 = _round4(x0, x1, _R0)
    x0, x1 = x0 + _KS2, x1 + (_KS0 + np.uint32(5))
    return x0 ^ x1


def _bits_to_uniform(bits):
    f = lax.bitcast_convert_type(
        lax.shift_right_logical(bits, jnp.full_like(bits, np.uint32(9)))
        | np.uint32(0x3F800000),
        jnp.float32,
    ) - jnp.float32(1.0)
    return f * (jnp.float32(1.0) - _TINY) + _TINY


def _tc_sample_block(logits_ref, idx_ref, x_ref):
    pid = pl.program_id(0)
    base = (pid * (ROWS * VOCAB)).astype(jnp.uint32)
    r = lax.broadcasted_iota(jnp.uint32, (ROWS, VOCAB), 0)
    c = lax.broadcasted_iota(jnp.uint32, (ROWS, VOCAB), 1)
    lin = base + r * np.uint32(VOCAB) + c

    u = _bits_to_uniform(_threefry_fold(lin))
    g = -jnp.log(-jnp.log(u))

    s = logits_ref[...] + g
    mx = jnp.max(s, axis=1, keepdims=True)
    col = lax.broadcasted_iota(jnp.int32, (ROWS, VOCAB), 1)
    idx = jnp.min(
        jnp.where(s == mx, col, jnp.int32(VOCAB)), axis=1, keepdims=True
    )
    idxf = idx.astype(jnp.float32)
    idx_ref[...] = idxf
    x_ref[...] = jnp.float32(-_A) + jnp.float32(_SCALE) * idxf


def _tc_sample(logits):
    grid = (TC_ROWS // ROWS,)
    return pl.pallas_call(
        _tc_sample_block,
        grid=grid,
        in_specs=[pl.BlockSpec((ROWS, VOCAB), lambda i: (i, 0))],
        out_specs=[
            pl.BlockSpec((ROWS, 1), lambda i: (i, 0)),
            pl.BlockSpec((ROWS, 1), lambda i: (i, 0)),
        ],
        out_shape=[
            jax.ShapeDtypeStruct((BATCH, 1), jnp.float32),
            jax.ShapeDtypeStruct((BATCH, 1), jnp.float32),
        ],
    )(logits)


NW = 32
RPW = SC_ROWS // NW
CH = 16
NCH = RPW // CH
IB = 4
NJ = VOCAB // (16 * IB)

_SQRT2 = np.float32(np.sqrt(2.0))
_LN2 = np.float32(np.log(2.0))
_LOG_COEF = tuple(
    np.float32(c)
    for c in (
        -0.0776459202170372, 0.12656806409358978, -0.13065096735954285,
        0.14209529757499695, -0.16633062064647675, 0.2000124156475067,
        -0.2500060498714447, 0.333333283662796, -0.4999999701976776,
    )
)


def _log_f32(v):
    bits = lax.bitcast_convert_type(v, jnp.int32)
    e = (bits >> jnp.int32(23)) - jnp.int32(127)
    m = lax.bitcast_convert_type(
        (bits & jnp.int32(0x7FFFFF)) | jnp.int32(0x3F800000), jnp.float32
    )
    big = m >= _SQRT2
    m = jnp.where(big, m * jnp.float32(0.5), m)
    e = jnp.where(big, e + jnp.int32(1), e)
    t = m - jnp.float32(1.0)
    p = jnp.full_like(t, _LOG_COEF[0])
    for c in _LOG_COEF[1:]:
        p = p * t + c
    return (t + (t * t) * p) + e.astype(jnp.float32) * _LN2


@functools.cache
def _make_sc_sample():
    mesh = plsc.VectorSubcoreMesh(core_axis_name="c", subcore_axis_name="s")
    return functools.partial(
        pl.kernel,
        out_type=[
            jax.ShapeDtypeStruct((SC_ROWS, 16), jnp.float32),
            jax.ShapeDtypeStruct((SC_ROWS, 16), jnp.int32),
        ],
        mesh=mesh,
        scratch_types=[
            pltpu.VMEM((CH, VOCAB), jnp.float32),
            pltpu.VMEM((CH, VOCAB), jnp.float32),
            pltpu.VMEM((RPW, 16), jnp.float32),
            pltpu.VMEM((RPW, 16), jnp.int32),
            pltpu.SemaphoreType.DMA,
            pltpu.SemaphoreType.DMA,
        ],
    )(_sc_sample_body)


def _sc_sample_body(logits_hbm, bv_hbm, bc_hbm, buf0, buf1, obv, obc, sem0, sem1):
    wid = lax.axis_index("s") * 2 + lax.axis_index("c")
    base = wid * RPW
    bufs = (buf0, buf1)
    sems = (sem0, sem1)
    iota_i = lax.iota(jnp.int32, 16)
    big = jnp.full((16,), 3.0e38, jnp.float32)
    zero_c = jnp.zeros((16,), jnp.int32)

    copies = [None] * NCH
    copies[0] = pltpu.async_copy(
        logits_hbm.at[pl.ds(TC_ROWS + base, CH)], bufs[0], sems[0]
    )

    for k in range(NCH):
        copies[k].wait()
        if k + 1 < NCH:
            copies[k + 1] = pltpu.async_copy(
                logits_hbm.at[pl.ds(TC_ROWS + base + (k + 1) * CH, CH)],
                bufs[(k + 1) % 2],
                sems[(k + 1) % 2],
            )
        buf = bufs[k % 2]

        def row_body(r, _, k=k, buf=buf):
            row_local = k * CH + r
            rowbase = (TC_ROWS + base + row_local) * jnp.int32(VOCAB)

            def col_body(jj, carry):
                bvs, bcs = carry
                nbvs, nbcs = [], []
                for t in range(IB):
                    col0 = jj * jnp.int32(16 * IB) + jnp.int32(t * 16)
                    col = col0 + iota_i
                    lvec = buf[r, pl.ds(col0, 16)]
                    lin = (rowbase + col).astype(jnp.uint32)
                    u = _bits_to_uniform(_threefry_fold(lin))
                    y = -_log_f32(u)
                    v = y * jnp.exp(-lvec)
                    upd = v < bvs[t]
                    nbvs.append(jnp.where(upd, v, bvs[t]))
                    nbcs.append(jnp.where(upd, col, bcs[t]))
                return tuple(nbvs), tuple(nbcs)

            bvs, bcs = lax.fori_loop(
                0, NJ, col_body, ((big,) * IB, (zero_c,) * IB)
            )
            bv, bc = bvs[0], bcs[0]
            for t in range(1, IB):
                upd = (bvs[t] < bv) | ((bvs[t] == bv) & (bcs[t] < bc))
                bv = jnp.where(upd, bvs[t], bv)
                bc = jnp.where(upd, bcs[t], bc)
            obv[row_local, :] = bv
            obc[row_local, :] = bc
            return 0

        lax.fori_loop(0, CH, row_body, 0)

    pltpu.sync_copy(obv, bv_hbm.at[pl.ds(base, RPW)])
    pltpu.sync_copy(obc, bc_hbm.at[pl.ds(base, RPW)])


MR = 512


def _merge_block(bv_ref, bc_ref, idx_in_ref, x_in_ref, idx_ref, x_ref):
    bv = bv_ref[...]
    bc = bc_ref[...]
    m = jnp.min(bv, axis=1, keepdims=True)
    idx = jnp.min(
        jnp.where(bv == m, bc, jnp.int32(VOCAB)), axis=1, keepdims=True
    )
    idxf = idx.astype(jnp.float32)
    idx_ref[...] = idxf
    x_ref[...] = jnp.float32(-_A) + jnp.float32(_SCALE) * idxf


def _sc_merge(bv, bc, idx_buf, x_buf):
    grid = (SC_ROWS // MR,)
    off = TC_ROWS // MR
    return pl.pallas_call(
        _merge_block,
        grid=grid,
        in_specs=[
            pl.BlockSpec((MR, 16), lambda i: (i, 0)),
            pl.BlockSpec((MR, 16), lambda i: (i, 0)),
            pl.BlockSpec((MR, 1), lambda i: (i + off, 0)),
            pl.BlockSpec((MR, 1), lambda i: (i + off, 0)),
        ],
        out_specs=[
            pl.BlockSpec((MR, 1), lambda i: (i + off, 0)),
            pl.BlockSpec((MR, 1), lambda i: (i + off, 0)),
        ],
        out_shape=[
            jax.ShapeDtypeStruct((BATCH, 1), jnp.float32),
            jax.ShapeDtypeStruct((BATCH, 1), jnp.float32),
        ],
        input_output_aliases={2: 0, 3: 1},
    )(bv, bc, idx_buf, x_buf)


@jax.jit
def kernel(logits):
    idx_tc, x_tc = _tc_sample(logits)
    bv, bc = _make_sc_sample()(logits)
    idx, x = _sc_merge(bv, bc, idx_tc, x_tc)
    return jnp.concatenate([idx, x], axis=-1)


# speedup vs baseline: 1.0267x; 1.0267x over previous
import functools

import jax
import jax.numpy as jnp
import numpy as np
from jax import lax
from jax.experimental import pallas as pl
from jax.experimental.pallas import tpu as pltpu
from jax.experimental.pallas import tpu_sc as plsc

BATCH = 16384
VOCAB = 1024
SNR = 10.0
_A = float(np.sqrt(10 ** (SNR / 10)))
_SCALE = (2.0 * _A) / (VOCAB - 1.0)

ROWS = 512
SC_ROWS = 3584
TC_ROWS = BATCH - SC_ROWS

_KS0 = np.uint32(0)
_KS1 = np.uint32(42)
_KS2 = np.uint32(int(_KS0) ^ int(_KS1) ^ 0x1BD11BDA)
_R0 = (13, 15, 26, 6)
_R1 = (17, 29, 16, 24)
_TINY = np.float32(np.finfo(np.float32).tiny)


def _rotl(x, r):
    return (x << np.uint32(r)) | (
        lax.shift_right_logical(x, jnp.full_like(x, np.uint32(32 - r)))
    )


def _round4(x0, x1, rots):
    for r in rots:
        x0 = x0 + x1
        x1 = _rotl(x1, r) ^ x0
    return x0, x1


def _threefry_fold(lin):
    x1i = lin + _KS1
    x0 = x1i
    x1 = _rotl(x1i, 13) ^ x1i
    x0, x1 = _round4(x0, x1, _R0[1:])
    x0, x1 = x0 + _KS1, x1 + (_KS2 + np.uint32(1))
    x0, x1 = _round4(x0, x1, _R1)
    x0, x1 = x0 + _KS2, x1 + (_KS0 + np.uint32(2))
    x0, x1 = _round4(x0, x1, _R0)
    x0, x1 = x0 + _KS0, x1 + (_KS1 + np.uint32(3))
    x0, x1 = _round4(x0, x1, _R1)
    x0, x1 = x0 + _KS1, x1 + (_KS2 + np.uint32(4))
    x0, x1 = _round4(x0, x1, _R0)
    x0, x1 = x0 + _KS2, x1 + (_KS0 + np.uint32(5))
    return x0 ^ x1


def _bits_to_uniform(bits):
    f = lax.bitcast_convert_type(
        lax.shift_right_logical(bits, jnp.full_like(bits, np.uint32(9)))
        | np.uint32(0x3F800000),
        jnp.float32,
    ) - jnp.float32(1.0)
    return f * (jnp.float32(1.0) - _TINY) + _TINY


def _tc_sample_block(logits_ref, out_ref):
    pid = pl.program_id(0)
    base = (pid * (ROWS * VOCAB)).astype(jnp.uint32)
    r = lax.broadcasted_iota(jnp.uint32, (ROWS, VOCAB), 0)
    c = lax.broadcasted_iota(jnp.uint32, (ROWS, VOCAB), 1)
    lin = base + r * np.uint32(VOCAB) + c

    u = _bits_to_uniform(_threefry_fold(lin))
    g = -jnp.log(-jnp.log(u))

    s = logits_ref[...] + g
    mx = jnp.max(s, axis=1, keepdims=True)
    col = lax.broadcasted_iota(jnp.int32, (ROWS, VOCAB), 1)
    idx = jnp.min(
        jnp.where(s == mx, col, jnp.int32(VOCAB)), axis=1, keepdims=True
    )
    idxf = idx.astype(jnp.float32)
    xv = jnp.float32(-_A) + jnp.float32(_SCALE) * idxf
    out_ref[...] = jnp.concatenate([idxf, xv], axis=1)


def _tc_sample(logits):
    grid = (TC_ROWS // ROWS,)
    return pl.pallas_call(
        _tc_sample_block,
        grid=grid,
        in_specs=[pl.BlockSpec((ROWS, VOCAB), lambda i: (i, 0))],
        out_specs=pl.BlockSpec((ROWS, 2), lambda i: (i, 0)),
        out_shape=jax.ShapeDtypeStruct((BATCH, 2), jnp.float32),
    )(logits)


NW = 32
RPW = SC_ROWS // NW
CH = 16
NCH = RPW // CH
IB = 4
NJ = VOCAB // (16 * IB)

_SQRT2 = np.float32(np.sqrt(2.0))
_LN2 = np.float32(np.log(2.0))
_LOG_COEF = tuple(
    np.float32(c)
    for c in (
        -0.0776459202170372, 0.12656806409358978, -0.13065096735954285,
        0.14209529757499695, -0.16633062064647675, 0.2000124156475067,
        -0.2500060498714447, 0.333333283662796, -0.4999999701976776,
    )
)


def _log_f32(v):
    bits = lax.bitcast_convert_type(v, jnp.int32)
    e = (bits >> jnp.int32(23)) - jnp.int32(127)
    m = lax.bitcast_convert_type(
        (bits & jnp.int32(0x7FFFFF)) | jnp.int32(0x3F800000), jnp.float32
    )
    big = m >= _SQRT2
    m = jnp.where(big, m * jnp.float32(0.5), m)
    e = jnp.where(big, e + jnp.int32(1), e)
    t = m - jnp.float32(1.0)
    p = jnp.full_like(t, _LOG_COEF[0])
    for c in _LOG_COEF[1:]:
        p = p * t + c
    return (t + (t * t) * p) + e.astype(jnp.float32) * _LN2


@functools.cache
def _make_sc_sample():
    mesh = plsc.VectorSubcoreMesh(core_axis_name="c", subcore_axis_name="s")
    return functools.partial(
        pl.kernel,
        out_type=[
            jax.ShapeDtypeStruct((SC_ROWS, 16), jnp.float32),
            jax.ShapeDtypeStruct((SC_ROWS, 16), jnp.int32),
        ],
        mesh=mesh,
        scratch_types=[
            pltpu.VMEM((CH, VOCAB), jnp.float32),
            pltpu.VMEM((CH, VOCAB), jnp.float32),
            pltpu.VMEM((RPW, 16), jnp.float32),
            pltpu.VMEM((RPW, 16), jnp.int32),
            pltpu.SemaphoreType.DMA,
            pltpu.SemaphoreType.DMA,
        ],
    )(_sc_sample_body)


def _sc_sample_body(logits_hbm, bv_hbm, bc_hbm, buf0, buf1, obv, obc, sem0, sem1):
    wid = lax.axis_index("s") * 2 + lax.axis_index("c")
    base = wid * RPW
    bufs = (buf0, buf1)
    sems = (sem0, sem1)
    iota_i = lax.iota(jnp.int32, 16)
    big = jnp.full((16,), 3.0e38, jnp.float32)
    zero_c = jnp.zeros((16,), jnp.int32)

    copies = [None] * NCH
    copies[0] = pltpu.async_copy(
        logits_hbm.at[pl.ds(TC_ROWS + base, CH)], bufs[0], sems[0]
    )

    for k in range(NCH):
        copies[k].wait()
        if k + 1 < NCH:
            copies[k + 1] = pltpu.async_copy(
                logits_hbm.at[pl.ds(TC_ROWS + base + (k + 1) * CH, CH)],
                bufs[(k + 1) % 2],
                sems[(k + 1) % 2],
            )
        buf = bufs[k % 2]

        def row_body(r, _, k=k, buf=buf):
            row_local = k * CH + r
            rowbase = (TC_ROWS + base + row_local) * jnp.int32(VOCAB)

            def col_body(jj, carry):
                bvs, bcs = carry
                nbvs, nbcs = [], []
                for t in range(IB):
                    col0 = jj * jnp.int32(16 * IB) + jnp.int32(t * 16)
                    col = col0 + iota_i
                    lvec = buf[r, pl.ds(col0, 16)]
                    lin = (rowbase + col).astype(jnp.uint32)
                    u = _bits_to_uniform(_threefry_fold(lin))
                    y = -_log_f32(u)
                    v = y * jnp.exp(-lvec)
                    upd = v < bvs[t]
                    nbvs.append(jnp.where(upd, v, bvs[t]))
                    nbcs.append(jnp.where(upd, col, bcs[t]))
                return tuple(nbvs), tuple(nbcs)

            bvs, bcs = lax.fori_loop(
                0, NJ, col_body, ((big,) * IB, (zero_c,) * IB)
            )
            bv, bc = bvs[0], bcs[0]
            for t in range(1, IB):
                upd = (bvs[t] < bv) | ((bvs[t] == bv) & (bcs[t] < bc))
                bv = jnp.where(upd, bvs[t], bv)
                bc = jnp.where(upd, bcs[t], bc)
            obv[row_local, :] = bv
            obc[row_local, :] = bc
            return 0

        lax.fori_loop(0, CH, row_body, 0)

    pltpu.sync_copy(obv, bv_hbm.at[pl.ds(base, RPW)])
    pltpu.sync_copy(obc, bc_hbm.at[pl.ds(base, RPW)])


MR = 512


def _merge_block(bv_ref, bc_ref, out_in_ref, out_ref):
    bv = bv_ref[...]
    bc = bc_ref[...]
    m = jnp.min(bv, axis=1, keepdims=True)
    idx = jnp.min(
        jnp.where(bv == m, bc, jnp.int32(VOCAB)), axis=1, keepdims=True
    )
    idxf = idx.astype(jnp.float32)
    xv = jnp.float32(-_A) + jnp.float32(_SCALE) * idxf
    out_ref[...] = jnp.concatenate([idxf, xv], axis=1)


def _sc_merge(bv, bc, out_buf):
    grid = (SC_ROWS // MR,)
    off = TC_ROWS // MR
    return pl.pallas_call(
        _merge_block,
        grid=grid,
        in_specs=[
            pl.BlockSpec((MR, 16), lambda i: (i, 0)),
            pl.BlockSpec((MR, 16), lambda i: (i, 0)),
            pl.BlockSpec((MR, 2), lambda i: (i + off, 0)),
        ],
        out_specs=pl.BlockSpec((MR, 2), lambda i: (i + off, 0)),
        out_shape=jax.ShapeDtypeStruct((BATCH, 2), jnp.float32),
        input_output_aliases={2: 0},
    )(bv, bc, out_buf)


@jax.jit
def kernel(logits):
    out_tc = _tc_sample(logits)
    bv, bc = _make_sc_sample()(logits)
    return _sc_merge(bv, bc, out_tc)
